# Initial kernel scaffold; baseline (speedup 1.0000x reference)
import jax
import jax.numpy as jnp
from jax.experimental import pallas as pl


def kernel(table, qlen, klen):
    raise NotImplementedError("write your pallas kernel here")


# speedup vs baseline: 89.2626x; 89.2626x over previous
import math

import jax
import jax.numpy as jnp
from jax.experimental import pallas as pl
from jax.experimental.pallas import tpu as pltpu

NUM_HEADS = 16
NUM_BUCKETS = 32
MAX_DISTANCE = 128
QLEN = 2048
KLEN = 2048
TQ = 256
NQB = QLEN // TQ
WWIN = 2 * KLEN


def _lookup_kernel(table_ref, w_ref):
    shape = (NUM_HEADS, NQB, 1, WWIN)
    qb = jax.lax.broadcasted_iota(jnp.int32, shape, 1)
    j = jax.lax.broadcasted_iota(jnp.int32, shape, 3)
    rel = KLEN + TQ * qb - j
    sign = rel < 0
    dist = jnp.abs(rel)
    max_exact = NUM_BUCKETS // 2
    is_small = dist < max_exact
    large_f = (
        jnp.log(dist.astype(jnp.float32) / max_exact + 1e-06)
        / math.log(MAX_DISTANCE / max_exact + 1e-06)
        * (NUM_BUCKETS - max_exact)
    )
    large_bucket = max_exact + large_f.astype(jnp.int32)
    large_bucket = jnp.minimum(large_bucket, NUM_BUCKETS - 1)
    buckets = jnp.where(is_small, dist, large_bucket)
    buckets = jnp.where(sign, buckets, buckets + max_exact)
    buckets = jnp.minimum(buckets, NUM_BUCKETS - 1)
    acc = jnp.zeros(shape, jnp.float32)
    for b in range(NUM_BUCKETS):
        tcol = table_ref[b, :][:, None, None, None]
        acc = jnp.where(buckets == b, tcol, acc)
    w_ref[...] = acc


def _expand_kernel(w_ref, out_ref):
    w = w_ref[0, 0, 0, :]
    a = jnp.broadcast_to(w[KLEN:2 * KLEN][None, :], (TQ, KLEN))
    b = jnp.broadcast_to(w[0:KLEN][None, :], (TQ, KLEN))
    rota = pltpu.roll(a, 0, 1, stride=1, stride_axis=0)
    rotb = pltpu.roll(b, 0, 1, stride=1, stride_axis=0)
    row = jax.lax.broadcasted_iota(jnp.int32, (TQ, KLEN), 0)
    col = jax.lax.broadcasted_iota(jnp.int32, (TQ, KLEN), 1)
    out_ref[0, :, :] = jnp.where(col >= row, rota, rotb)


def kernel(table, qlen, klen):
    w_all = pl.pallas_call(
        _lookup_kernel,
        out_shape=jax.ShapeDtypeStruct((NUM_HEADS, NQB, 1, WWIN), jnp.float32),
    )(table)
    out = pl.pallas_call(
        _expand_kernel,
        grid=(NUM_HEADS, NQB),
        in_specs=[pl.BlockSpec((1, 1, 1, WWIN), lambda h, q: (h, q, 0, 0))],
        out_specs=pl.BlockSpec((1, TQ, KLEN), lambda h, q: (h, q, 0)),
        out_shape=jax.ShapeDtypeStruct((NUM_HEADS, QLEN, KLEN), jnp.float32),
    )(w_all)
    return out


# speedup vs baseline: 146.9280x; 1.6460x over previous
import math

import jax
import jax.numpy as jnp
from jax.experimental import pallas as pl
from jax.experimental.pallas import tpu as pltpu

NUM_HEADS = 16
NUM_BUCKETS = 32
MAX_DISTANCE = 128
QLEN = 2048
KLEN = 2048
TQ = 512
NQB = QLEN // TQ
WWIN = 2 * KLEN


def _lookup_kernel(table_ref, w_ref):
    shape = (NUM_HEADS, NQB, 1, WWIN)
    qb = jax.lax.broadcasted_iota(jnp.int32, shape, 1)
    j = jax.lax.broadcasted_iota(jnp.int32, shape, 3)
    rel = KLEN + TQ * qb - j
    sign = rel < 0
    dist = jnp.abs(rel)
    max_exact = NUM_BUCKETS // 2
    is_small = dist < max_exact
    large_f = (
        jnp.log(dist.astype(jnp.float32) / max_exact + 1e-06)
        / math.log(MAX_DISTANCE / max_exact + 1e-06)
        * (NUM_BUCKETS - max_exact)
    )
    large_bucket = max_exact + large_f.astype(jnp.int32)
    large_bucket = jnp.minimum(large_bucket, NUM_BUCKETS - 1)
    buckets = jnp.where(is_small, dist, large_bucket)
    buckets = jnp.where(sign, buckets, buckets + max_exact)
    buckets = jnp.minimum(buckets, NUM_BUCKETS - 1)
    acc = jnp.zeros(shape, jnp.float32)
    for b in range(NUM_BUCKETS):
        tcol = table_ref[b, :][:, None, None, None]
        acc = jnp.where(buckets == b, tcol, acc)
    w_ref[...] = acc


def _expand_kernel(w_ref, out_ref):
    w = jnp.broadcast_to(w_ref[0, 0, 0, :][None, :], (TQ, WWIN))
    rolled = pltpu.roll(w, 0, 1, stride=1, stride_axis=0)
    out_ref[0, :, :] = rolled[:, KLEN:]


def kernel(table, qlen, klen):
    w_all = pl.pallas_call(
        _lookup_kernel,
        out_shape=jax.ShapeDtypeStruct((NUM_HEADS, NQB, 1, WWIN), jnp.float32),
    )(table)
    out = pl.pallas_call(
        _expand_kernel,
        grid=(NUM_HEADS, NQB),
        in_specs=[pl.BlockSpec((1, 1, 1, WWIN), lambda h, q: (h, q, 0, 0))],
        out_specs=pl.BlockSpec((1, TQ, KLEN), lambda h, q: (h, q, 0)),
        out_shape=jax.ShapeDtypeStruct((NUM_HEADS, QLEN, KLEN), jnp.float32),
        compiler_params=pltpu.CompilerParams(
            dimension_semantics=("parallel", "parallel"),
        ),
    )(w_all)
    return out


# speedup vs baseline: 170.0487x; 1.1574x over previous
import math

import jax
import jax.numpy as jnp
from jax.experimental import pallas as pl
from jax.experimental.pallas import tpu as pltpu

NUM_HEADS = 16
NUM_BUCKETS = 32
MAX_DISTANCE = 128
QLEN = 2048
KLEN = 2048
TQ = 1024
NQB = QLEN // TQ
WWIN = 2 * KLEN


def _lookup_kernel(table_ref, w_ref):
    shape = (NUM_HEADS, NQB, 1, WWIN)
    qb = jax.lax.broadcasted_iota(jnp.int32, shape, 1)
    j = jax.lax.broadcasted_iota(jnp.int32, shape, 3)
    rel = KLEN + TQ * qb - j
    sign = rel < 0
    dist = jnp.abs(rel)
    max_exact = NUM_BUCKETS // 2
    is_small = dist < max_exact
    large_f = (
        jnp.log(dist.astype(jnp.float32) / max_exact + 1e-06)
        / math.log(MAX_DISTANCE / max_exact + 1e-06)
        * (NUM_BUCKETS - max_exact)
    )
    large_bucket = max_exact + large_f.astype(jnp.int32)
    large_bucket = jnp.minimum(large_bucket, NUM_BUCKETS - 1)
    buckets = jnp.where(is_small, dist, large_bucket)
    buckets = jnp.where(sign, buckets, buckets + max_exact)
    buckets = jnp.minimum(buckets, NUM_BUCKETS - 1)
    acc = jnp.zeros(shape, jnp.float32)
    for b in range(NUM_BUCKETS):
        tcol = table_ref[b, :][:, None, None, None]
        acc = jnp.where(buckets == b, tcol, acc)
    w_ref[...] = acc


def _expand_kernel(w_ref, out_ref):
    w = jnp.broadcast_to(w_ref[0, 0, 0, :][None, :], (TQ, WWIN))
    rolled = pltpu.roll(w, 0, 1, stride=1, stride_axis=0)
    out_ref[0, :, :] = rolled[:, KLEN:]


def kernel(table, qlen, klen):
    w_all = pl.pallas_call(
        _lookup_kernel,
        out_shape=jax.ShapeDtypeStruct((NUM_HEADS, NQB, 1, WWIN), jnp.float32),
    )(table)
    out = pl.pallas_call(
        _expand_kernel,
        grid=(NUM_HEADS, NQB),
        in_specs=[pl.BlockSpec((1, 1, 1, WWIN), lambda h, q: (h, q, 0, 0))],
        out_specs=pl.BlockSpec((1, TQ, KLEN), lambda h, q: (h, q, 0)),
        out_shape=jax.ShapeDtypeStruct((NUM_HEADS, QLEN, KLEN), jnp.float32),
        compiler_params=pltpu.CompilerParams(
            dimension_semantics=("parallel", "parallel"),
        ),
    )(w_all)
    return out


# speedup vs baseline: 178.1641x; 1.0477x over previous
import math

import jax
import jax.numpy as jnp
from jax.experimental import pallas as pl
from jax.experimental.pallas import tpu as pltpu

NUM_HEADS = 16
NUM_BUCKETS = 32
MAX_DISTANCE = 128
QLEN = 2048
KLEN = 2048
TQ = 2048
NQB = QLEN // TQ
WWIN = 2 * KLEN


def _lookup_kernel(table_ref, w_ref):
    shape = (NUM_HEADS, NQB, 1, WWIN)
    qb = jax.lax.broadcasted_iota(jnp.int32, shape, 1)
    j = jax.lax.broadcasted_iota(jnp.int32, shape, 3)
    rel = KLEN + TQ * qb - j
    sign = rel < 0
    dist = jnp.abs(rel)
    max_exact = NUM_BUCKETS // 2
    is_small = dist < max_exact
    large_f = (
        jnp.log(dist.astype(jnp.float32) / max_exact + 1e-06)
        / math.log(MAX_DISTANCE / max_exact + 1e-06)
        * (NUM_BUCKETS - max_exact)
    )
    large_bucket = max_exact + large_f.astype(jnp.int32)
    large_bucket = jnp.minimum(large_bucket, NUM_BUCKETS - 1)
    buckets = jnp.where(is_small, dist, large_bucket)
    buckets = jnp.where(sign, buckets, buckets + max_exact)
    buckets = jnp.minimum(buckets, NUM_BUCKETS - 1)
    acc = jnp.zeros(shape, jnp.float32)
    for b in range(NUM_BUCKETS):
        tcol = table_ref[b, :][:, None, None, None]
        acc = jnp.where(buckets == b, tcol, acc)
    w_ref[...] = acc


def _expand_kernel(w_ref, out_ref):
    wrow = w_ref[0, 0, 0, :][None, :]
    ch = 512
    for c in range(TQ // ch):
        w = jnp.broadcast_to(wrow, (ch, WWIN))
        rolled = pltpu.roll(w, c * ch, 1, stride=1, stride_axis=0)
        out_ref[0, c * ch:(c + 1) * ch, :] = rolled[:, KLEN:]


def kernel(table, qlen, klen):
    w_all = pl.pallas_call(
        _lookup_kernel,
        out_shape=jax.ShapeDtypeStruct((NUM_HEADS, NQB, 1, WWIN), jnp.float32),
    )(table)
    out = pl.pallas_call(
        _expand_kernel,
        grid=(NUM_HEADS, NQB),
        in_specs=[pl.BlockSpec((1, 1, 1, WWIN), lambda h, q: (h, q, 0, 0))],
        out_specs=pl.BlockSpec((1, TQ, KLEN), lambda h, q: (h, q, 0)),
        out_shape=jax.ShapeDtypeStruct((NUM_HEADS, QLEN, KLEN), jnp.float32),
        compiler_params=pltpu.CompilerParams(
            dimension_semantics=("parallel", "parallel"),
        ),
    )(w_all)
    return out
